# baseline (speedup 1.0000x reference)
import functools

import jax
import jax.numpy as jnp
from jax.experimental import pallas as pl
from jax.experimental.pallas import tpu as pltpu

_R = 256
_PIPE = 96
_MAN = _R - _PIPE


def _issue_block(idx_ref, conf_hbm, buf, sem, step):
    base = step * _R

    def issue_one(k, carry):
        row = idx_ref[base + k]
        pltpu.make_async_copy(
            conf_hbm.at[pl.ds(row, 1), :],
            buf.at[pl.ds(k, 1), :],
            sem,
        ).start()
        return carry

    jax.lax.fori_loop(0, _MAN, issue_one, 0, unroll=8)


def _loss_part(x, g):
    m = jnp.max(x, axis=1, keepdims=True)
    lse = m + jnp.log(jnp.sum(jnp.exp(x - m), axis=1, keepdims=True))
    s1 = jnp.sum(g, axis=1, keepdims=True)
    d = jnp.sum(x * g, axis=1, keepdims=True)
    return lse * s1 - d


def _body(idx_ref, x_ref, conf_hbm, *refs):
    pipe_refs = refs[:_PIPE]
    loss_ref, acc_ref, buf, sem = refs[_PIPE:]
    i = pl.program_id(0)
    nsteps = pl.num_programs(0)
    par = jax.lax.rem(i, 2)
    nxt = jax.lax.rem(i + 1, 2)

    @pl.when(i == 0)
    def _():
        _issue_block(idx_ref, conf_hbm, buf.at[0], sem.at[0], 0)

    @pl.when(i + 1 < nsteps)
    def _():
        _issue_block(idx_ref, conf_hbm, buf.at[nxt], sem.at[nxt], i + 1)

    pltpu.make_async_copy(
        conf_hbm.at[pl.ds(0, _MAN), :], buf.at[par], sem.at[par]
    ).wait()

    loss1 = _loss_part(x_ref[pl.ds(0, _MAN), :], buf[par])
    loss_ref[pl.ds(0, _MAN), :] = loss1

    g2 = jnp.concatenate([r[0] for r in pipe_refs], axis=0)
    loss2 = _loss_part(x_ref[pl.ds(_MAN, _PIPE), :], g2)
    loss_ref[pl.ds(_MAN, _PIPE), :] = loss2

    @pl.when(i == 0)
    def _():
        acc_ref[...] = jnp.zeros_like(acc_ref)

    total = acc_ref[...] + (jnp.sum(loss1) + jnp.sum(loss2)).reshape(1, 1)
    acc_ref[...] = total

    @pl.when(i == nsteps - 1)
    def _():
        acc_ref[...] = total / (nsteps * _R)


def kernel(outputs, index, confidence):
    B, C = outputs.shape
    N = confidence.shape[0]
    G = B // _R
    conf3 = confidence.reshape(N, 1, C)
    pipe_specs = [
        pl.BlockSpec(
            (1, 1, C),
            functools.partial(
                lambda i, idx, j=0: (idx[i * _R + _MAN + j], 0, 0), j=j
            ),
        )
        for j in range(_PIPE)
    ]
    grid_spec = pltpu.PrefetchScalarGridSpec(
        num_scalar_prefetch=1,
        grid=(G,),
        in_specs=[
            pl.BlockSpec((_R, C), lambda i, idx: (i, 0)),
            pl.BlockSpec(memory_space=pl.ANY),
        ]
        + pipe_specs,
        out_specs=[
            pl.BlockSpec((_R, 1), lambda i, idx: (i, 0)),
            pl.BlockSpec((1, 1), lambda i, idx: (0, 0)),
        ],
        scratch_shapes=[
            pltpu.VMEM((2, _MAN, C), jnp.float32),
            pltpu.SemaphoreType.DMA((2,)),
        ],
    )
    loss2, acc = pl.pallas_call(
        _body,
        grid_spec=grid_spec,
        out_shape=[
            jax.ShapeDtypeStruct((B, 1), jnp.float32),
            jax.ShapeDtypeStruct((1, 1), jnp.float32),
        ],
    )(index, outputs, confidence, *([conf3] * _PIPE))
    return (acc[0, 0], loss2.reshape(B))


# speedup vs baseline: 3.0372x; 3.0372x over previous
import jax
import jax.numpy as jnp
from jax.experimental import pallas as pl
from jax.experimental.pallas import tpu as pltpu

_R = 1024


def _issue_block(idx_ref, conf_hbm, buf, sem, step):
    base = step * _R

    def issue_one(k, carry):
        row = idx_ref[base + k]
        pltpu.make_async_copy(
            conf_hbm.at[pl.ds(row, 1), :],
            buf.at[pl.ds(k, 1), :],
            sem,
        ).start()
        return carry

    jax.lax.fori_loop(0, _R, issue_one, 0, unroll=8)


def _wait_block(conf_hbm, buf, sem):
    pltpu.make_async_copy(conf_hbm.at[pl.ds(0, _R), :], buf, sem).wait()


def _body(idx_ref, x_ref, conf_hbm, loss_ref, acc_ref, buf, sem):
    i = pl.program_id(0)
    nsteps = pl.num_programs(0)
    par = jax.lax.rem(i, 2)
    nxt = jax.lax.rem(i + 1, 2)

    @pl.when(i == 0)
    def _():
        _issue_block(idx_ref, conf_hbm, buf.at[0], sem.at[0], 0)

    @pl.when(i + 1 < nsteps)
    def _():
        _issue_block(idx_ref, conf_hbm, buf.at[nxt], sem.at[nxt], i + 1)

    _wait_block(conf_hbm, buf.at[par], sem.at[par])

    x = x_ref[...]
    g = buf[par]
    m = jnp.max(x, axis=1, keepdims=True)
    lse = m + jnp.log(jnp.sum(jnp.exp(x - m), axis=1, keepdims=True))
    s1 = jnp.sum(g, axis=1, keepdims=True)
    d = jnp.sum(x * g, axis=1, keepdims=True)
    loss = lse * s1 - d
    loss_ref[...] = loss

    @pl.when(i == 0)
    def _():
        acc_ref[...] = jnp.zeros_like(acc_ref)

    total = acc_ref[...] + jnp.sum(loss).reshape(1, 1)
    acc_ref[...] = total

    @pl.when(i == nsteps - 1)
    def _():
        acc_ref[...] = total / (nsteps * _R)


def kernel(outputs, index, confidence):
    B, C = outputs.shape
    G = B // _R
    grid_spec = pltpu.PrefetchScalarGridSpec(
        num_scalar_prefetch=1,
        grid=(G,),
        in_specs=[
            pl.BlockSpec((_R, C), lambda i, idx: (i, 0)),
            pl.BlockSpec(memory_space=pl.ANY),
        ],
        out_specs=[
            pl.BlockSpec((_R, 1), lambda i, idx: (i, 0)),
            pl.BlockSpec((1, 1), lambda i, idx: (0, 0)),
        ],
        scratch_shapes=[
            pltpu.VMEM((2, _R, C), jnp.float32),
            pltpu.SemaphoreType.DMA((2,)),
        ],
    )
    loss2, acc = pl.pallas_call(
        _body,
        grid_spec=grid_spec,
        out_shape=[
            jax.ShapeDtypeStruct((B, 1), jnp.float32),
            jax.ShapeDtypeStruct((1, 1), jnp.float32),
        ],
    )(index, outputs, confidence)
    return (acc[0, 0], loss2.reshape(B))
